# baseline (speedup 1.0000x reference)
import functools

import jax
import jax.numpy as jnp
import numpy as np
from jax import lax
from jax.experimental import pallas as pl
from jax.experimental.pallas import tpu as pltpu
from jax.experimental.pallas import tpu_sc as plsc

B = 4096
F = 26
D = 16
FD = F * D
V = 1000000

NC = 2
NS = 16
NW = NC * NS
IDX_PER_W = B * F // NW
CHUNK = 128
NCH = IDX_PER_W // CHUNK


@functools.cache
def _get_sc_gather():
    mesh = plsc.VectorSubcoreMesh(core_axis_name="c", subcore_axis_name="s")

    @functools.partial(
        pl.kernel,
        mesh=mesh,
        compiler_params=pltpu.CompilerParams(use_tc_tiling_on_sc=False),
        out_type=[
            jax.ShapeDtypeStruct((D, NW, NCH, CHUNK), jnp.float32),
            jax.ShapeDtypeStruct((NW, NCH, CHUNK), jnp.float32),
        ],
        scratch_types=[
            pltpu.VMEM((D, NCH, CHUNK), jnp.int32),
            pltpu.VMEM((D, NCH, CHUNK), jnp.float32),
            pltpu.VMEM((NCH, CHUNK), jnp.float32),
            pltpu.SemaphoreType.DMA,
            pltpu.SemaphoreType.DMA,
        ],
    )
    def _sc_gather(idx_hbm, emb_hbm, lw_hbm, out_emb, out_lw,
                   idx_v, rows_v, lwv_v, sem_e, sem_l):
        wid = lax.axis_index("s") * NC + lax.axis_index("c")
        pltpu.sync_copy(idx_hbm.at[:, wid], idx_v)
        lw_descs = []
        for j in range(NCH):
            lw_descs.append(
                pltpu.async_copy(lw_hbm.at[idx_v.at[0, j]], lwv_v.at[j], sem_l))

        def _per_component(d, carry):
            descs = []
            for j in range(NCH):
                descs.append(
                    pltpu.async_copy(emb_hbm.at[idx_v.at[d, j]],
                                     rows_v.at[d, j], sem_e))
            for de in descs:
                de.wait()
            return carry

        lax.fori_loop(0, D, _per_component, 0)
        for de in lw_descs:
            de.wait()
        pltpu.sync_copy(rows_v, out_emb.at[:, wid])
        pltpu.sync_copy(lwv_v, out_lw.at[wid])

    return _sc_gather


def _tc_body(x_ref, m_ref, lw_ref, bias_ref, o_ref):
    x = x_ref[...]
    t = jnp.dot(m_ref[...], x, preferred_element_type=jnp.float32)
    second = jnp.sum(x * t, axis=0, keepdims=True)
    first = jnp.sum(lw_ref[...], axis=0, keepdims=True)
    o_ref[...] = first + second + bias_ref[0, 0]


_I, _J = np.triu_indices(F, 1)


def kernel(inputs, embedding_weights, field_weights, linear_weights, bias_weight):
    idx_f = inputs.T.reshape(NW, NCH, CHUNK)
    idx16 = idx_f[None] + (jnp.arange(D, dtype=jnp.int32) * V).reshape(D, 1, 1, 1)
    emb_flat = embedding_weights.T.reshape(V * D)

    gathered, lw_g = _get_sc_gather()(idx16, emb_flat, linear_weights)
    x2 = gathered.reshape(FD, B)
    lw2 = lw_g.reshape(F, B)

    w = jnp.zeros((F, F), jnp.float32).at[_I, _J].set(field_weights[:, 0])
    m = jnp.kron(jnp.eye(D, dtype=jnp.float32), 0.5 * (w + w.T))

    out = pl.pallas_call(
        _tc_body,
        out_shape=jax.ShapeDtypeStruct((1, B), jnp.float32),
    )(x2, m, lw2, bias_weight.reshape(1, 1))
    return out.reshape(B, 1)


# speedup vs baseline: 2.4141x; 2.4141x over previous
import functools

import jax
import jax.numpy as jnp
import numpy as np
from jax import lax
from jax.experimental import pallas as pl
from jax.experimental.pallas import tpu as pltpu
from jax.experimental.pallas import tpu_sc as plsc

B = 4096
F = 26
D = 16
NPAIR = F * (F - 1) // 2

NC = 2
NS = 16
NW = NC * NS
SAMP_PER_W = B // NW
IDX_PER_W = SAMP_PER_W * F
CHUNK = 128
NCH = IDX_PER_W // CHUNK
WLEN = 336


@functools.cache
def _get_sc_kernel():
    mesh = plsc.VectorSubcoreMesh(core_axis_name="c", subcore_axis_name="s")

    @functools.partial(
        pl.kernel,
        mesh=mesh,
        compiler_params=pltpu.CompilerParams(use_tc_tiling_on_sc=False,
                                             needs_layout_passes=False),
        out_type=jax.ShapeDtypeStruct((NW, SAMP_PER_W), jnp.float32),
        scratch_types=[
            pltpu.VMEM((NCH, CHUNK), jnp.int32),
            pltpu.VMEM((IDX_PER_W, D), jnp.float32),
            pltpu.VMEM((IDX_PER_W,), jnp.float32),
            pltpu.VMEM((SAMP_PER_W, D), jnp.float32),
            pltpu.VMEM((SAMP_PER_W,), jnp.float32),
            pltpu.VMEM((WLEN,), jnp.float32),
            pltpu.SemaphoreType.DMA,
            pltpu.SemaphoreType.DMA,
            pltpu.SemaphoreType.DMA,
        ],
    )
    def _sc_fwfm(idx_hbm, emb_hbm, lw_hbm, wtab_hbm, out_hbm,
                 idx_v, rows_v, lwv_v, p_v, out_v, wtab_v,
                 sem_e, sem_l, sem_w):
        wid = lax.axis_index("s") * NC + lax.axis_index("c")
        pltpu.async_copy(wtab_hbm, wtab_v, sem_w).wait()
        pltpu.sync_copy(idx_hbm.at[wid], idx_v)
        descs = []
        for j in range(NCH):
            descs.append(pltpu.async_copy(emb_hbm.at[idx_v.at[j]],
                                          rows_v.at[pl.ds(j * CHUNK, CHUNK)],
                                          sem_e))
            descs.append(pltpu.async_copy(lw_hbm.at[idx_v.at[j]],
                                          lwv_v.at[pl.ds(j * CHUNK, CHUNK)],
                                          sem_l))
        for de in descs:
            de.wait()

        wvec = [wtab_v[pl.ds(i * D, D)] for i in range(WLEN // D)]
        ws = [wvec[k // D][k % D] for k in range(NPAIR)]
        bias = wvec[NPAIR // D][NPAIR % D]

        def _sample(l, carry):
            base = l * F
            e = [rows_v[base + f, :] for f in range(F)]
            p = None
            k = 0
            for f in range(F - 1):
                tmp = ws[k] * e[f + 1]
                k += 1
                for g in range(f + 2, F):
                    tmp = tmp + ws[k] * e[g]
                    k += 1
                contrib = e[f] * tmp
                p = contrib if p is None else p + contrib
            p_v[l, :] = p
            return carry

        lax.fori_loop(0, SAMP_PER_W, _sample, 0)
        lane = lax.iota(jnp.int32, D)
        for grp in range(SAMP_PER_W // D):
            rows = grp * D + lane
            acc = plsc.load_gather(p_v, [rows, jnp.zeros((D,), jnp.int32)])
            for d in range(1, D):
                acc = acc + plsc.load_gather(
                    p_v, [rows, jnp.full((D,), d, jnp.int32)])
            lwrow = rows * F
            for f in range(F):
                acc = acc + plsc.load_gather(lwv_v, [lwrow + f])
            out_v[pl.ds(grp * D, D)] = acc + bias

        pltpu.sync_copy(out_v, out_hbm.at[wid])

    return _sc_fwfm


def kernel(inputs, embedding_weights, field_weights, linear_weights, bias_weight):
    idx2 = inputs.reshape(NW, NCH, CHUNK)
    wtab = jnp.zeros((WLEN,), jnp.float32)
    wtab = wtab.at[:NPAIR].set(field_weights[:, 0])
    wtab = wtab.at[NPAIR].set(bias_weight)
    out = _get_sc_kernel()(idx2, embedding_weights, linear_weights, wtab)
    return out.reshape(B, 1)


# speedup vs baseline: 2.6810x; 1.1106x over previous
import functools

import jax
import jax.numpy as jnp
import numpy as np
from jax import lax
from jax.experimental import pallas as pl
from jax.experimental.pallas import tpu as pltpu
from jax.experimental.pallas import tpu_sc as plsc

B = 4096
F = 26
D = 16
NPAIR = F * (F - 1) // 2
V = 1000000
VG = V // 8
GW = 128

NC = 2
NS = 16
NW = NC * NS
SAMP_PER_W = B // NW
IDX_PER_W = SAMP_PER_W * F
CHUNK = 128
NCH = IDX_PER_W // CHUNK
NCHP = 32
WROWS = 336


@functools.cache
def _get_sc_kernel():
    mesh = plsc.VectorSubcoreMesh(core_axis_name="c", subcore_axis_name="s")

    @functools.partial(
        pl.kernel,
        mesh=mesh,
        compiler_params=pltpu.CompilerParams(use_tc_tiling_on_sc=True,
                                             needs_layout_passes=False),
        out_type=jax.ShapeDtypeStruct((B,), jnp.float32),
        scratch_types=[
            pltpu.VMEM((NCHP, CHUNK), jnp.int32),
            pltpu.VMEM((NCH * CHUNK,), jnp.int32),
            pltpu.VMEM((2, CHUNK, GW), jnp.float32),
            pltpu.VMEM((IDX_PER_W * D,), jnp.float32),
            pltpu.VMEM((IDX_PER_W,), jnp.float32),
            pltpu.VMEM((SAMP_PER_W * D,), jnp.float32),
            pltpu.VMEM((SAMP_PER_W,), jnp.float32),
            pltpu.VMEM((WROWS * D,), jnp.float32),
            pltpu.SemaphoreType.DMA,
            pltpu.SemaphoreType.DMA,
            pltpu.SemaphoreType.DMA,
        ],
    )
    def _sc_fwfm(idx_hbm, emb_hbm, lw_hbm, wbro_hbm, out_hbm,
                 orig_v, gidx_v, grp_v, rows_v, lwv_v, p_v, out_v, wb_v,
                 sem_e, sem_l, sem_w):
        wid = lax.axis_index("s") * NC + lax.axis_index("c")
        pltpu.async_copy(wbro_hbm, wb_v, sem_w).wait()
        pltpu.sync_copy(idx_hbm.at[pl.ds(wid * NCHP, NCHP)], orig_v)

        lw_descs = []
        for j in range(NCH):
            lw_descs.append(pltpu.async_copy(
                lw_hbm.at[orig_v.at[j]], lwv_v.at[pl.ds(j * CHUNK, CHUNK)],
                sem_l))

        def _gidx_body(j, carry):
            for c in range(CHUNK // D):
                s = pl.ds(c * D, D)
                gidx_v[pl.ds(j * CHUNK + c * D, D)] = (
                    lax.shift_right_logical(orig_v[j, s], 3))
            return carry

        lax.fori_loop(0, NCH, _gidx_body, 0)

        lane = lax.iota(jnp.int32, D)

        def _gather(j, buf):
            return pltpu.async_copy(
                emb_hbm.at[gidx_v.at[pl.ds(j * CHUNK, CHUNK)]],
                grp_v.at[buf], sem_e)

        _gather(0, 0)
        _gather(1, 1)

        def _pipe_body(j, carry):
            buf = j & 1
            pltpu.make_async_copy(emb_hbm.at[pl.ds(0, CHUNK)],
                                  grp_v.at[buf], sem_e).wait()
            bufv = jnp.full((D,), 0, jnp.int32) + buf
            jv = jnp.full((D,), 0, jnp.int32) + j
            for r0 in range(0, CHUNK, D):
                rvec = lane + r0
                ovec = plsc.load_gather(orig_v, [jv, rvec])
                ovec = (ovec & 7) * D
                dst_base = (j * CHUNK + rvec) * D
                for d in range(D):
                    vals = plsc.load_gather(grp_v, [bufv, rvec, ovec + d])
                    plsc.store_scatter(rows_v, [dst_base + d], vals)

            @pl.when(j + 2 < NCH)
            def _():
                _gather(j + 2, buf)

            return carry

        lax.fori_loop(0, NCH, _pipe_body, 0)

        for de in lw_descs:
            de.wait()

        bias_v = wb_v[pl.ds(NPAIR * D, D)]

        def _sample2(l2, carry):
            base0 = (2 * l2) * F * D
            base1 = base0 + F * D
            e0 = [rows_v[pl.ds(base0 + f * D, D)] for f in range(F)]
            e1 = [rows_v[pl.ds(base1 + f * D, D)] for f in range(F)]
            p0 = None
            p1 = None
            k = 0
            for f in range(F - 1):
                w = wb_v[pl.ds(k * D, D)]
                t0 = w * e0[f + 1]
                t1 = w * e1[f + 1]
                k += 1
                for g in range(f + 2, F):
                    w = wb_v[pl.ds(k * D, D)]
                    t0 = t0 + w * e0[g]
                    t1 = t1 + w * e1[g]
                    k += 1
                c0 = e0[f] * t0
                c1 = e1[f] * t1
                p0 = c0 if p0 is None else p0 + c0
                p1 = c1 if p1 is None else p1 + c1
            p_v[pl.ds((2 * l2) * D, D)] = p0
            p_v[pl.ds((2 * l2 + 1) * D, D)] = p1
            return carry

        lax.fori_loop(0, SAMP_PER_W // 2, _sample2, 0)

        for grp in range(SAMP_PER_W // D):
            rows = (grp * D + lane) * D
            acc = bias_v
            for d in range(D):
                acc = acc + plsc.load_gather(p_v, [rows + d])
            lwrow = (grp * D + lane) * F
            for f in range(F):
                acc = acc + plsc.load_gather(lwv_v, [lwrow + f])
            out_v[pl.ds(grp * D, D)] = acc

        pltpu.sync_copy(out_v, out_hbm.at[pl.ds(wid * SAMP_PER_W, SAMP_PER_W)])

    return _sc_fwfm


def kernel(inputs, embedding_weights, field_weights, linear_weights, bias_weight):
    flat = inputs.reshape(B * F)
    idx3 = jnp.pad(flat.reshape(NW, NCH, CHUNK),
                   ((0, 0), (0, NCHP - NCH), (0, 0)))
    emb128 = embedding_weights.reshape(VG, GW)

    wtab = jnp.zeros((WROWS,), jnp.float32)
    wtab = wtab.at[:NPAIR].set(field_weights[:, 0])
    wtab = wtab.at[NPAIR].set(bias_weight)
    wbro = jnp.broadcast_to(wtab[:, None], (WROWS, D)).reshape(WROWS * D)

    out = _get_sc_kernel()(idx3.reshape(NW * NCHP, CHUNK),
                           emb128, linear_weights, wbro)
    return out.reshape(B, 1)


# speedup vs baseline: 4.9084x; 1.8308x over previous
import functools

import jax
import jax.numpy as jnp
import numpy as np
from jax import lax
from jax.experimental import pallas as pl
from jax.experimental.pallas import tpu as pltpu
from jax.experimental.pallas import tpu_sc as plsc

B = 4096
F = 26
D = 16
NPAIR = F * (F - 1) // 2
V = 1000000
VG = V // 8
GW = 128

NC = 2
NS = 16
NW = NC * NS
SAMP_PER_W = B // NW
IDX_PER_W = SAMP_PER_W * F
CHUNK = 128
NCH = IDX_PER_W // CHUNK
NCHP = 32
WROWS = 336


NBLK = V // GW
BPW = NBLK // NW
NEXTRA = NBLK - BPW * NW
VTAIL = V - NBLK * GW


@functools.cache
def _get_sc_transpose():
    mesh = plsc.VectorSubcoreMesh(core_axis_name="c", subcore_axis_name="s")

    @functools.partial(
        pl.kernel,
        mesh=mesh,
        compiler_params=pltpu.CompilerParams(use_tc_tiling_on_sc=True,
                                             needs_layout_passes=False),
        out_type=jax.ShapeDtypeStruct((VG, GW), jnp.float32),
        scratch_types=[
            pltpu.VMEM((2, D, GW), jnp.float32),
            pltpu.VMEM((2, D, GW), jnp.float32),
            pltpu.SemaphoreType.DMA,
            pltpu.SemaphoreType.DMA,
        ],
    )
    def _sc_tr(embt_hbm, out_hbm, src_v, dst_v, sem_i, sem_o):
        wid = lax.axis_index("s") * NC + lax.axis_index("c")
        lane16 = lax.iota(jnp.int32, D) * D

        def _fetch(k, buf):
            c0 = pl.multiple_of((wid * BPW + k) * GW, GW)
            return pltpu.async_copy(
                embt_hbm.at[:, pl.ds(c0, GW)], src_v.at[buf], sem_i)

        def _transpose(buf, nrow):
            bv = jnp.full((D,), 0, jnp.int32) + buf
            for d in range(D):
                for r0 in range(0, nrow, D):
                    vals = src_v[buf, d, pl.ds(r0, D)]
                    p = lane16 + (r0 * D + d)
                    plsc.store_scatter(
                        dst_v, [bv, lax.shift_right_logical(p, 7), p & 127],
                        vals)

        _fetch(0, 0)

        def _panel(k, carry):
            buf = k & 1
            pltpu.make_async_copy(embt_hbm.at[:, pl.ds(0, GW)],
                                  src_v.at[buf], sem_i).wait()

            @pl.when(k + 1 < BPW)
            def _():
                _fetch(k + 1, 1 - (k & 1))

            @pl.when(k >= 2)
            def _():
                pltpu.make_async_copy(dst_v.at[0], out_hbm.at[pl.ds(0, D)],
                                      sem_o).wait()

            _transpose(buf, GW)
            g0 = pl.multiple_of((wid * BPW + k) * D, D)
            pltpu.async_copy(dst_v.at[buf], out_hbm.at[pl.ds(g0, D)], sem_o)
            return carry

        lax.fori_loop(0, BPW, _panel, 0)
        pltpu.make_async_copy(dst_v.at[0], out_hbm.at[pl.ds(0, D)], sem_o).wait()
        pltpu.make_async_copy(dst_v.at[0], out_hbm.at[pl.ds(0, D)], sem_o).wait()

        @pl.when(wid < NEXTRA)
        def _():
            c0 = pl.multiple_of((NW * BPW + wid) * GW, GW)
            pltpu.async_copy(embt_hbm.at[:, pl.ds(c0, GW)],
                             src_v.at[0], sem_i).wait()
            _transpose(0, GW)
            g0 = pl.multiple_of((NW * BPW + wid) * D, D)
            pltpu.async_copy(dst_v.at[0], out_hbm.at[pl.ds(g0, D)],
                             sem_o).wait()


    return _sc_tr


@functools.cache
def _get_sc_kernel():
    mesh = plsc.VectorSubcoreMesh(core_axis_name="c", subcore_axis_name="s")

    @functools.partial(
        pl.kernel,
        mesh=mesh,
        compiler_params=pltpu.CompilerParams(use_tc_tiling_on_sc=True,
                                             needs_layout_passes=False),
        out_type=jax.ShapeDtypeStruct((B,), jnp.float32),
        scratch_types=[
            pltpu.VMEM((NCHP, CHUNK), jnp.int32),
            pltpu.VMEM((NCH * CHUNK,), jnp.int32),
            pltpu.VMEM((2, CHUNK, GW), jnp.float32),
            pltpu.VMEM((IDX_PER_W * D,), jnp.float32),
            pltpu.VMEM((IDX_PER_W,), jnp.float32),
            pltpu.VMEM((SAMP_PER_W * D,), jnp.float32),
            pltpu.VMEM((SAMP_PER_W,), jnp.float32),
            pltpu.VMEM((WROWS * D,), jnp.float32),
            pltpu.VMEM((VTAIL * D,), jnp.float32),
            pltpu.SemaphoreType.DMA,
            pltpu.SemaphoreType.DMA,
            pltpu.SemaphoreType.DMA,
        ],
    )
    def _sc_fwfm(idx_hbm, emb_hbm, lw_hbm, wbro_hbm, tail_hbm, out_hbm,
                 orig_v, gidx_v, grp_v, rows_v, lwv_v, p_v, out_v, wb_v,
                 tail_v, sem_e, sem_l, sem_w):
        wid = lax.axis_index("s") * NC + lax.axis_index("c")
        pltpu.async_copy(wbro_hbm, wb_v, sem_w).wait()
        pltpu.async_copy(tail_hbm, tail_v, sem_w).wait()
        pltpu.sync_copy(idx_hbm.at[pl.ds(wid * NCHP, NCHP)], orig_v)

        lw_descs = []
        for j in range(NCH):
            lw_descs.append(pltpu.async_copy(
                lw_hbm.at[orig_v.at[j]], lwv_v.at[pl.ds(j * CHUNK, CHUNK)],
                sem_l))

        def _gidx_body(j, carry):
            for c in range(CHUNK // D):
                s = pl.ds(c * D, D)
                gidx_v[pl.ds(j * CHUNK + c * D, D)] = jnp.minimum(
                    lax.shift_right_logical(orig_v[j, s], 3), VG - VTAIL // 8 - 1)
            return carry

        lax.fori_loop(0, NCH, _gidx_body, 0)

        lane = lax.iota(jnp.int32, D)

        def _gather(j, buf):
            return pltpu.async_copy(
                emb_hbm.at[gidx_v.at[pl.ds(j * CHUNK, CHUNK)]],
                grp_v.at[buf], sem_e)

        _gather(0, 0)
        _gather(1, 1)

        def _pipe_body(j, carry):
            buf = j & 1
            pltpu.make_async_copy(emb_hbm.at[pl.ds(0, CHUNK)],
                                  grp_v.at[buf], sem_e).wait()
            bufv = jnp.full((D,), 0, jnp.int32) + buf
            jv = jnp.full((D,), 0, jnp.int32) + j
            for r0 in range(0, CHUNK, D):
                rvec = lane + r0
                origv = plsc.load_gather(orig_v, [jv, rvec])
                ovec = (origv & 7) * D
                tsel = origv >= V - VTAIL
                tbase = jnp.maximum(origv - (V - VTAIL), 0) * D
                dst_base = (j * CHUNK + rvec) * D
                for d in range(D):
                    vals = plsc.load_gather(grp_v, [bufv, rvec, ovec + d])
                    tv = plsc.load_gather(tail_v, [tbase + d])
                    vals = jnp.where(tsel, tv, vals)
                    plsc.store_scatter(rows_v, [dst_base + d], vals)

            @pl.when(j + 2 < NCH)
            def _():
                _gather(j + 2, buf)

            return carry

        lax.fori_loop(0, NCH, _pipe_body, 0)

        for de in lw_descs:
            de.wait()

        bias_v = wb_v[pl.ds(NPAIR * D, D)]

        def _sample2(l2, carry):
            base0 = (2 * l2) * F * D
            base1 = base0 + F * D
            e0 = [rows_v[pl.ds(base0 + f * D, D)] for f in range(F)]
            e1 = [rows_v[pl.ds(base1 + f * D, D)] for f in range(F)]
            p0 = None
            p1 = None
            k = 0
            for f in range(F - 1):
                w = wb_v[pl.ds(k * D, D)]
                t0 = w * e0[f + 1]
                t1 = w * e1[f + 1]
                k += 1
                for g in range(f + 2, F):
                    w = wb_v[pl.ds(k * D, D)]
                    t0 = t0 + w * e0[g]
                    t1 = t1 + w * e1[g]
                    k += 1
                c0 = e0[f] * t0
                c1 = e1[f] * t1
                p0 = c0 if p0 is None else p0 + c0
                p1 = c1 if p1 is None else p1 + c1
            p_v[pl.ds((2 * l2) * D, D)] = p0
            p_v[pl.ds((2 * l2 + 1) * D, D)] = p1
            return carry

        lax.fori_loop(0, SAMP_PER_W // 2, _sample2, 0)

        for grp in range(SAMP_PER_W // D):
            rows = (grp * D + lane) * D
            acc = bias_v
            for d in range(D):
                acc = acc + plsc.load_gather(p_v, [rows + d])
            lwrow = (grp * D + lane) * F
            for f in range(F):
                acc = acc + plsc.load_gather(lwv_v, [lwrow + f])
            out_v[pl.ds(grp * D, D)] = acc

        pltpu.sync_copy(out_v, out_hbm.at[pl.ds(wid * SAMP_PER_W, SAMP_PER_W)])

    return _sc_fwfm


def kernel(inputs, embedding_weights, field_weights, linear_weights, bias_weight):
    flat = inputs.reshape(B * F)
    idx3 = jnp.pad(flat.reshape(NW, NCH, CHUNK),
                   ((0, 0), (0, NCHP - NCH), (0, 0)))
    emb128 = _get_sc_transpose()(embedding_weights.T)

    wtab = jnp.zeros((WROWS,), jnp.float32)
    wtab = wtab.at[:NPAIR].set(field_weights[:, 0])
    wtab = wtab.at[NPAIR].set(bias_weight)
    wbro = jnp.broadcast_to(wtab[:, None], (WROWS, D)).reshape(WROWS * D)

    tail = embedding_weights[V - VTAIL:].reshape(VTAIL * D)
    out = _get_sc_kernel()(idx3.reshape(NW * NCHP, CHUNK),
                           emb128, linear_weights, wbro, tail)
    return out.reshape(B, 1)


# speedup vs baseline: 5.3627x; 1.0925x over previous
import functools

import jax
import jax.numpy as jnp
import numpy as np
from jax import lax
from jax.experimental import pallas as pl
from jax.experimental.pallas import tpu as pltpu
from jax.experimental.pallas import tpu_sc as plsc

B = 4096
F = 26
D = 16
NPAIR = F * (F - 1) // 2
V = 1000000
VG = V // 8
GW = 128

NC = 2
NS = 16
NW = NC * NS
SAMP_PER_W = B // NW
IDX_PER_W = SAMP_PER_W * F
CHUNK = 128
NCH = IDX_PER_W // CHUNK
NCHP = 32
WROWS = 336


NBLK = V // GW
BPW = NBLK // NW
NEXTRA = NBLK - BPW * NW
VTAIL = V - NBLK * GW


@functools.cache
def _get_sc_transpose():
    mesh = plsc.VectorSubcoreMesh(core_axis_name="c", subcore_axis_name="s")

    @functools.partial(
        pl.kernel,
        mesh=mesh,
        compiler_params=pltpu.CompilerParams(use_tc_tiling_on_sc=True,
                                             needs_layout_passes=False),
        out_type=jax.ShapeDtypeStruct((VG, GW), jnp.float32),
        scratch_types=[
            pltpu.VMEM((4, D, GW), jnp.float32),
            pltpu.VMEM((2, D, GW), jnp.float32),
            pltpu.SemaphoreType.DMA,
            pltpu.SemaphoreType.DMA,
        ],
    )
    def _sc_tr(embt_hbm, out_hbm, src_v, dst_v, sem_i, sem_o):
        wid = lax.axis_index("s") * NC + lax.axis_index("c")
        lane16 = lax.iota(jnp.int32, D) * D

        def _fetch(k, buf):
            c0 = pl.multiple_of((wid * BPW + k) * GW, GW)
            return pltpu.async_copy(
                embt_hbm.at[:, pl.ds(c0, GW)], src_v.at[buf], sem_i)

        def _transpose(buf, nrow, dbuf):
            bv = jnp.full((D,), 0, jnp.int32) + dbuf
            for d in range(D):
                for r0 in range(0, nrow, D):
                    vals = src_v[buf, d, pl.ds(r0, D)]
                    p = lane16 + (r0 * D + d)
                    plsc.store_scatter(
                        dst_v, [bv, lax.shift_right_logical(p, 7), p & 127],
                        vals)

        _fetch(0, 0)
        _fetch(1, 1)
        _fetch(2, 2)

        def _panel(k, carry):
            buf = k & 3
            dbuf = k & 1
            pltpu.make_async_copy(embt_hbm.at[:, pl.ds(0, GW)],
                                  src_v.at[0], sem_i).wait()

            @pl.when(k + 3 < BPW)
            def _():
                _fetch(k + 3, (k + 3) & 3)

            @pl.when(k >= 2)
            def _():
                pltpu.make_async_copy(dst_v.at[0], out_hbm.at[pl.ds(0, D)],
                                      sem_o).wait()

            _transpose(buf, GW, dbuf)
            g0 = pl.multiple_of((wid * BPW + k) * D, D)
            pltpu.async_copy(dst_v.at[dbuf], out_hbm.at[pl.ds(g0, D)], sem_o)
            return carry

        lax.fori_loop(0, BPW, _panel, 0)
        pltpu.make_async_copy(dst_v.at[0], out_hbm.at[pl.ds(0, D)], sem_o).wait()
        pltpu.make_async_copy(dst_v.at[0], out_hbm.at[pl.ds(0, D)], sem_o).wait()

        @pl.when(wid < NEXTRA)
        def _():
            c0 = pl.multiple_of((NW * BPW + wid) * GW, GW)
            pltpu.async_copy(embt_hbm.at[:, pl.ds(c0, GW)],
                             src_v.at[0], sem_i).wait()
            _transpose(0, GW, 0)
            g0 = pl.multiple_of((NW * BPW + wid) * D, D)
            pltpu.async_copy(dst_v.at[0], out_hbm.at[pl.ds(g0, D)],
                             sem_o).wait()


    return _sc_tr


@functools.cache
def _get_sc_kernel():
    mesh = plsc.VectorSubcoreMesh(core_axis_name="c", subcore_axis_name="s")

    @functools.partial(
        pl.kernel,
        mesh=mesh,
        compiler_params=pltpu.CompilerParams(use_tc_tiling_on_sc=True,
                                             needs_layout_passes=False),
        out_type=jax.ShapeDtypeStruct((B,), jnp.float32),
        scratch_types=[
            pltpu.VMEM((NCHP, CHUNK), jnp.int32),
            pltpu.VMEM((NCH * CHUNK,), jnp.int32),
            pltpu.VMEM((2, CHUNK, GW), jnp.float32),
            pltpu.VMEM((IDX_PER_W * D,), jnp.float32),
            pltpu.VMEM((IDX_PER_W,), jnp.float32),
            pltpu.VMEM((SAMP_PER_W * D,), jnp.float32),
            pltpu.VMEM((SAMP_PER_W,), jnp.float32),
            pltpu.VMEM((WROWS * D,), jnp.float32),
            pltpu.VMEM((VTAIL * D,), jnp.float32),
            pltpu.SemaphoreType.DMA,
            pltpu.SemaphoreType.DMA,
            pltpu.SemaphoreType.DMA,
        ],
    )
    def _sc_fwfm(idx_hbm, emb_hbm, lw_hbm, wbro_hbm, tail_hbm, out_hbm,
                 orig_v, gidx_v, grp_v, rows_v, lwv_v, p_v, out_v, wb_v,
                 tail_v, sem_e, sem_l, sem_w):
        wid = lax.axis_index("s") * NC + lax.axis_index("c")
        pltpu.async_copy(wbro_hbm, wb_v, sem_w).wait()
        pltpu.async_copy(tail_hbm, tail_v, sem_w).wait()
        pltpu.sync_copy(idx_hbm.at[pl.ds(wid * NCHP, NCHP)], orig_v)

        lw_descs = []
        for j in range(NCH):
            lw_descs.append(pltpu.async_copy(
                lw_hbm.at[orig_v.at[j]], lwv_v.at[pl.ds(j * CHUNK, CHUNK)],
                sem_l))

        def _gidx_body(j, carry):
            for c in range(CHUNK // D):
                s = pl.ds(c * D, D)
                gidx_v[pl.ds(j * CHUNK + c * D, D)] = jnp.minimum(
                    lax.shift_right_logical(orig_v[j, s], 3), VG - VTAIL // 8 - 1)
            return carry

        lax.fori_loop(0, NCH, _gidx_body, 0)

        lane = lax.iota(jnp.int32, D)

        def _gather(j, buf):
            return pltpu.async_copy(
                emb_hbm.at[gidx_v.at[pl.ds(j * CHUNK, CHUNK)]],
                grp_v.at[buf], sem_e)

        _gather(0, 0)
        _gather(1, 1)

        def _pipe_body(j, carry):
            buf = j & 1
            pltpu.make_async_copy(emb_hbm.at[pl.ds(0, CHUNK)],
                                  grp_v.at[buf], sem_e).wait()
            bufv = jnp.full((D,), 0, jnp.int32) + buf
            jv = jnp.full((D,), 0, jnp.int32) + j
            for r0 in range(0, CHUNK, D):
                rvec = lane + r0
                origv = plsc.load_gather(orig_v, [jv, rvec])
                ovec = (origv & 7) * D
                tsel = origv >= V - VTAIL
                tbase = jnp.maximum(origv - (V - VTAIL), 0) * D
                dst_base = (j * CHUNK + rvec) * D
                for d in range(D):
                    vals = plsc.load_gather(grp_v, [bufv, rvec, ovec + d])
                    tv = plsc.load_gather(tail_v, [tbase + d])
                    vals = jnp.where(tsel, tv, vals)
                    plsc.store_scatter(rows_v, [dst_base + d], vals)

            @pl.when(j + 2 < NCH)
            def _():
                _gather(j + 2, buf)

            return carry

        lax.fori_loop(0, NCH, _pipe_body, 0)

        for de in lw_descs:
            de.wait()

        bias_v = wb_v[pl.ds(NPAIR * D, D)]

        def _sample2(l2, carry):
            base0 = (2 * l2) * F * D
            base1 = base0 + F * D
            e0 = [rows_v[pl.ds(base0 + f * D, D)] for f in range(F)]
            e1 = [rows_v[pl.ds(base1 + f * D, D)] for f in range(F)]
            p0 = None
            p1 = None
            k = 0
            for f in range(F - 1):
                w = wb_v[pl.ds(k * D, D)]
                t0 = w * e0[f + 1]
                t1 = w * e1[f + 1]
                k += 1
                for g in range(f + 2, F):
                    w = wb_v[pl.ds(k * D, D)]
                    t0 = t0 + w * e0[g]
                    t1 = t1 + w * e1[g]
                    k += 1
                c0 = e0[f] * t0
                c1 = e1[f] * t1
                p0 = c0 if p0 is None else p0 + c0
                p1 = c1 if p1 is None else p1 + c1
            p_v[pl.ds((2 * l2) * D, D)] = p0
            p_v[pl.ds((2 * l2 + 1) * D, D)] = p1
            return carry

        lax.fori_loop(0, SAMP_PER_W // 2, _sample2, 0)

        for grp in range(SAMP_PER_W // D):
            rows = (grp * D + lane) * D
            acc = bias_v
            for d in range(D):
                acc = acc + plsc.load_gather(p_v, [rows + d])
            lwrow = (grp * D + lane) * F
            for f in range(F):
                acc = acc + plsc.load_gather(lwv_v, [lwrow + f])
            out_v[pl.ds(grp * D, D)] = acc

        pltpu.sync_copy(out_v, out_hbm.at[pl.ds(wid * SAMP_PER_W, SAMP_PER_W)])

    return _sc_fwfm


def kernel(inputs, embedding_weights, field_weights, linear_weights, bias_weight):
    flat = inputs.reshape(B * F)
    idx3 = jnp.pad(flat.reshape(NW, NCH, CHUNK),
                   ((0, 0), (0, NCHP - NCH), (0, 0)))
    emb128 = _get_sc_transpose()(embedding_weights.T)

    wtab = jnp.zeros((WROWS,), jnp.float32)
    wtab = wtab.at[:NPAIR].set(field_weights[:, 0])
    wtab = wtab.at[NPAIR].set(bias_weight)
    wbro = jnp.broadcast_to(wtab[:, None], (WROWS, D)).reshape(WROWS * D)

    tail = embedding_weights[V - VTAIL:].reshape(VTAIL * D)
    out = _get_sc_kernel()(idx3.reshape(NW * NCHP, CHUNK),
                           emb128, linear_weights, wbro, tail)
    return out.reshape(B, 1)
